# baseline (speedup 1.0000x reference)
import functools

import jax
import jax.numpy as jnp
from jax import lax
from jax.experimental import pallas as pl
from jax.experimental.pallas import tpu as pltpu
from jax.experimental.pallas import tpu_sc as plsc

_VOCAB = 100000
_D = 128
_B = 16384
_NEG = 10

_NC = 2
_NS = 16
_NW = _NC * _NS

_CH = 128
_TC_CH = _B // (_NW * _CH)
_NG_ROWS = _B * _NEG
_NG_CH = _NG_ROWS // (_NW * _CH)


def _make_kernel():
    mesh = plsc.VectorSubcoreMesh(core_axis_name="c", subcore_axis_name="s")

    @functools.partial(
        pl.kernel,
        mesh=mesh,
        out_type=(
            jax.ShapeDtypeStruct((_B, _D), jnp.float32),
            jax.ShapeDtypeStruct((_B, _D), jnp.float32),
            jax.ShapeDtypeStruct((_NEG, _B, _D), jnp.float32),
        ),
        scratch_types=[
            pltpu.VMEM((2 * _TC_CH + _NG_CH, _CH), jnp.int32),
            pltpu.VMEM((7, _CH, _D), jnp.float32),
            pltpu.SemaphoreType.DMA,
            pltpu.SemaphoreType.DMA,
            pltpu.SemaphoreType.DMA,
            pltpu.SemaphoreType.DMA,
            pltpu.SemaphoreType.DMA,
            pltpu.SemaphoreType.DMA,
            pltpu.SemaphoreType.DMA,
            pltpu.SemaphoreType.DMA,
            pltpu.SemaphoreType.DMA,
            pltpu.SemaphoreType.DMA,
            pltpu.SemaphoreType.DMA,
            pltpu.SemaphoreType.DMA,
            pltpu.SemaphoreType.DMA,
            pltpu.SemaphoreType.DMA,
        ],
    )
    def nsamp(t_hbm, c_hbm, n_hbm, w_hbm, out_t, out_c, out_n,
              idx_v, bufs, g0, g1, g2, g3, g4, g5, g6, s0, s1, s2, s3, s4, s5, s6):
        wid = lax.axis_index("s") * _NC + lax.axis_index("c")

        ic0 = pltpu.make_async_copy(t_hbm.at[pl.ds(wid * _TC_CH, _TC_CH)],
                                    idx_v.at[pl.ds(0, _TC_CH)], s0)
        ic1 = pltpu.make_async_copy(c_hbm.at[pl.ds(wid * _TC_CH, _TC_CH)],
                                    idx_v.at[pl.ds(_TC_CH, _TC_CH)], s0)
        ic2 = pltpu.make_async_copy(n_hbm.at[pl.ds(wid * _NG_CH, _NG_CH)],
                                    idx_v.at[pl.ds(2 * _TC_CH, _NG_CH)], s0)
        ic0.start()
        ic1.start()
        ic2.start()
        ic0.wait()
        ic1.wait()
        ic2.wait()

        gsems = (g0, g1, g2, g3, g4, g5, g6)
        ssems = (s0, s1, s2, s3, s4, s5, s6)

        def g_copy(ci, b):
            return pltpu.make_async_copy(
                w_hbm.at[idx_v.at[ci]], bufs.at[b], gsems[b])

        def s_copy(dst_slice, b):
            return pltpu.make_async_copy(bufs.at[b], dst_slice, ssems[b])

        def slice2d(out_ref):
            return lambda row: out_ref.at[pl.ds(row, _CH)]

        def slice3d(out_ref):
            return lambda row: out_ref.at[row // _B, pl.ds(row % _B, _CH)]

        NB, LD = 7, 3
        NCH = 2 * _TC_CH + _NG_CH

        dst_t = slice2d(out_t)
        dst_c = slice2d(out_c)
        dst_n = slice3d(out_n)
        row_t = wid * (_TC_CH * _CH)
        row_n = wid * (_NG_CH * _CH)

        def dst_for(k):
            if isinstance(k, int) and k < _TC_CH:
                return dst_t(row_t + k * _CH)
            if isinstance(k, int) and k < 2 * _TC_CH:
                return dst_c(row_t + (k - _TC_CH) * _CH)
            return dst_n(row_n + (k - 2 * _TC_CH) * _CH)

        def step(j, b, refill, br, swait):
            g_copy(j, b).wait()
            s_copy(dst_for(j), b).start()
            if refill:
                if swait:
                    s_copy(dst_n(row_n), br).wait()
                g_copy(j + LD, br).start()

        for j in range(LD):
            g_copy(j, j).start()
        for j in range(2 * _TC_CH):
            step(j, j % NB, True, (j + LD) % NB, j + LD >= NB)
        lo = 2 * _TC_CH
        hi = NCH - LD
        n_mid = ((hi - lo) // NB) * NB

        @pl.loop(lo, lo + n_mid, step=NB)
        def _main(j0):
            for d in range(NB):
                b = (lo + d) % NB
                step(j0 + d, b, True, (b + LD) % NB, True)

        for j in range(lo + n_mid, hi):
            step(j, j % NB, True, (j + LD) % NB, True)
        for j in range(hi, NCH):
            step(j, j % NB, False, 0, False)
        for j in range(NCH - NB, NCH):
            s_copy(dst_n(row_n), j % NB).wait()

    return nsamp


_gather_fused = _make_kernel()


def kernel(target, context, negative_samples, W):
    t2 = target.astype(jnp.int32).reshape(_B // _CH, _CH)
    c2 = context.astype(jnp.int32).reshape(_B // _CH, _CH)
    n2 = negative_samples.astype(jnp.int32).T.reshape(_NG_ROWS // _CH, _CH)
    out_t, out_c, out_n = _gather_fused(t2, c2, n2, W)
    return (out_t, out_c, out_n.transpose(1, 0, 2))


# speedup vs baseline: 1.0027x; 1.0027x over previous
import functools

import jax
import jax.numpy as jnp
from jax import lax
from jax.experimental import pallas as pl
from jax.experimental.pallas import tpu as pltpu
from jax.experimental.pallas import tpu_sc as plsc

_VOCAB = 100000
_D = 128
_B = 16384
_NEG = 10

_NC = 2
_NS = 16
_NW = _NC * _NS

_CH = 128
_TC_CH = _B // (_NW * _CH)
_NG_ROWS = _B * _NEG
_NG_CH = _NG_ROWS // (_NW * _CH)


def _make_kernel():
    mesh = plsc.VectorSubcoreMesh(core_axis_name="c", subcore_axis_name="s")

    @functools.partial(
        pl.kernel,
        mesh=mesh,
        out_type=(
            jax.ShapeDtypeStruct((_B, _D), jnp.float32),
            jax.ShapeDtypeStruct((_B, _D), jnp.float32),
            jax.ShapeDtypeStruct((_NEG, _B, _D), jnp.float32),
        ),
        scratch_types=[
            pltpu.VMEM((2 * _TC_CH + _NG_CH, _CH), jnp.int32),
            pltpu.VMEM((7, _CH, _D), jnp.float32),
            pltpu.SemaphoreType.DMA,
            pltpu.SemaphoreType.DMA,
            pltpu.SemaphoreType.DMA,
            pltpu.SemaphoreType.DMA,
            pltpu.SemaphoreType.DMA,
            pltpu.SemaphoreType.DMA,
            pltpu.SemaphoreType.DMA,
            pltpu.SemaphoreType.DMA,
            pltpu.SemaphoreType.DMA,
            pltpu.SemaphoreType.DMA,
            pltpu.SemaphoreType.DMA,
            pltpu.SemaphoreType.DMA,
            pltpu.SemaphoreType.DMA,
            pltpu.SemaphoreType.DMA,
        ],
    )
    def nsamp(t_hbm, c_hbm, n_hbm, w_hbm, out_t, out_c, out_n,
              idx_v, bufs, g0, g1, g2, g3, g4, g5, g6, s0, s1, s2, s3, s4, s5, s6):
        wid = lax.axis_index("s") * _NC + lax.axis_index("c")

        ic0 = pltpu.make_async_copy(t_hbm.at[pl.ds(wid * _TC_CH, _TC_CH)],
                                    idx_v.at[pl.ds(0, _TC_CH)], s0)
        ic1 = pltpu.make_async_copy(c_hbm.at[pl.ds(wid * _TC_CH, _TC_CH)],
                                    idx_v.at[pl.ds(_TC_CH, _TC_CH)], s0)
        ic2 = pltpu.make_async_copy(n_hbm.at[pl.ds(wid * _NG_CH, _NG_CH)],
                                    idx_v.at[pl.ds(2 * _TC_CH, _NG_CH)], s0)
        ic0.start()
        ic1.start()
        ic2.start()
        ic0.wait()
        ic1.wait()
        ic2.wait()

        gsems = (g0, g1, g2, g3, g4, g5, g6)
        ssems = (s0, s1, s2, s3, s4, s5, s6)

        def g_copy(ci, b):
            return pltpu.make_async_copy(
                w_hbm.at[idx_v.at[ci]], bufs.at[b], gsems[b])

        def s_copy(dst_slice, b):
            return pltpu.make_async_copy(bufs.at[b], dst_slice, ssems[b])

        def slice2d(out_ref):
            return lambda row: out_ref.at[pl.ds(row, _CH)]

        def slice3d(out_ref):
            return lambda row: out_ref.at[row // _B, pl.ds(row % _B, _CH)]

        NB, LD = 7, 4
        NCH = 2 * _TC_CH + _NG_CH

        dst_t = slice2d(out_t)
        dst_c = slice2d(out_c)
        dst_n = slice3d(out_n)
        row_t = wid * (_TC_CH * _CH)
        row_n = wid * (_NG_CH * _CH)

        def dst_for(k):
            if isinstance(k, int) and k < _TC_CH:
                return dst_t(row_t + k * _CH)
            if isinstance(k, int) and k < 2 * _TC_CH:
                return dst_c(row_t + (k - _TC_CH) * _CH)
            return dst_n(row_n + (k - 2 * _TC_CH) * _CH)

        def step(j, b, refill, br, swait):
            g_copy(j, b).wait()
            s_copy(dst_for(j), b).start()
            if refill:
                if swait:
                    s_copy(dst_n(row_n), br).wait()
                g_copy(j + LD, br).start()

        for j in range(LD):
            g_copy(j, j).start()
        for j in range(2 * _TC_CH):
            step(j, j % NB, True, (j + LD) % NB, j + LD >= NB)
        lo = 2 * _TC_CH
        hi = NCH - LD
        n_mid = ((hi - lo) // NB) * NB

        @pl.loop(lo, lo + n_mid, step=NB)
        def _main(j0):
            for d in range(NB):
                b = (lo + d) % NB
                step(j0 + d, b, True, (b + LD) % NB, True)

        for j in range(lo + n_mid, hi):
            step(j, j % NB, True, (j + LD) % NB, True)
        for j in range(hi, NCH):
            step(j, j % NB, False, 0, False)
        for j in range(NCH - NB, NCH):
            s_copy(dst_n(row_n), j % NB).wait()

    return nsamp


_gather_fused = _make_kernel()


def kernel(target, context, negative_samples, W):
    t2 = target.astype(jnp.int32).reshape(_B // _CH, _CH)
    c2 = context.astype(jnp.int32).reshape(_B // _CH, _CH)
    n2 = negative_samples.astype(jnp.int32).T.reshape(_NG_ROWS // _CH, _CH)
    out_t, out_c, out_n = _gather_fused(t2, c2, n2, W)
    return (out_t, out_c, out_n.transpose(1, 0, 2))
